# Initial kernel scaffold; baseline (speedup 1.0000x reference)
import jax
import jax.numpy as jnp
from jax.experimental import pallas as pl


def kernel(x, emb_tables, linear_table, bias):
    raise NotImplementedError("write your pallas kernel here")


# speedup vs baseline: 31.3627x; 31.3627x over previous
import functools

import jax
import jax.numpy as jnp
from jax import lax
from jax.experimental import pallas as pl
from jax.experimental.pallas import tpu as pltpu
from jax.experimental.pallas import tpu_sc as plsc

F = 26
V = 1000
D = 16
B = 4096
TOTAL = F * V

NC, NS = 2, 16
NW = NC * NS
BPW = B // NW
XW = BPW * F
NIDX = F * F
NPAD = 688


def _ffm_body(x_hbm, tab_hbm, lin16_hbm, bias_hbm, out_hbm,
              x_v, bias_v, idx0, idx1, rows0, rows1,
              lidx0, lidx1, lrows0, lrows1, out_v,
              sem0, sem1):
    wid = lax.axis_index("s") * NC + lax.axis_index("c")
    base = wid * BPW

    pltpu.sync_copy(x_hbm.at[pl.ds(base * F, XW)], x_v.at[pl.ds(0, XW)])
    pltpu.sync_copy(bias_hbm, bias_v)

    iota = lax.iota(jnp.int32, 16)
    off_lo = iota * V
    off_hi = jnp.where(iota < 10, (iota + 16) * V, 0)

    x_v[pl.ds(XW, 16)] = jnp.zeros((16,), jnp.int32)
    idx0[pl.ds(672, 16)] = jnp.zeros((16,), jnp.int32)
    idx1[pl.ds(672, 16)] = jnp.zeros((16,), jnp.int32)

    def lane_sum(v):
        for sh in (8, 4, 2, 1):
            perm = jnp.bitwise_xor(iota, sh)
            g = lax.gather(
                v, perm[:, None],
                lax.GatherDimensionNumbers(offset_dims=(),
                                           collapsed_slice_dims=(0,),
                                           start_index_map=(0,)),
                (1,), mode=lax.GatherScatterMode.PROMISE_IN_BOUNDS)
            v = v + g
        return v

    def xo_chunks(s):
        xl = x_v[pl.ds(s * F, 16)] + off_lo
        xh = x_v[pl.ds(s * F + 16, 16)] + off_hi
        return xl, xh

    def build_idx(s, idx_ref):
        xl, xh = xo_chunks(s)
        for m in range(F):
            idx_ref[pl.ds(m * F, 16)] = xl + m * TOTAL
            idx_ref[pl.ds(m * F + 16, 16)] = xh + m * TOTAL
        return xl, xh

    def start_gathers(s, idx_ref, lidx_ref, rows_ref, lrows_ref, sem):
        xl, xh = build_idx(s, idx_ref)
        lidx_ref[pl.ds(0, 16)] = xl
        lidx_ref[pl.ds(16, 16)] = xh
        pltpu.make_async_copy(tab_hbm.at[idx_ref], rows_ref, sem).start()
        pltpu.make_async_copy(lin16_hbm.at[lidx_ref], lrows_ref, sem).start()

    def wait_gathers(idx_ref, lidx_ref, rows_ref, lrows_ref, sem):
        pltpu.make_async_copy(tab_hbm.at[idx_ref], rows_ref, sem).wait()
        pltpu.make_async_copy(lin16_hbm.at[lidx_ref], lrows_ref, sem).wait()

    def compute(s, rows_ref, lrows_ref, zv):
        acc = jnp.zeros((16,), jnp.float32)
        for i in range(F):
            for j in range(i + 1, F):
                acc = acc + rows_ref[j * F + i] * rows_ref[i * F + j]
        for f in range(F):
            acc = acc + lrows_ref[f]
        return jnp.where(iota == lax.rem(s, 16), lane_sum(acc), zv)

    start_gathers(0, idx0, lidx0, rows0, lrows0, sem0)

    def body(k, zv):
        s = 2 * k
        start_gathers(s + 1, idx1, lidx1, rows1, lrows1, sem1)
        wait_gathers(idx0, lidx0, rows0, lrows0, sem0)
        zv = compute(s, rows0, lrows0, zv)

        @pl.when(k < BPW // 2 - 1)
        def _():
            start_gathers(s + 2, idx0, lidx0, rows0, lrows0, sem0)

        wait_gathers(idx1, lidx1, rows1, lrows1, sem1)
        zv = compute(s + 1, rows1, lrows1, zv)

        @pl.when(lax.rem(k, 8) == 7)
        def _():
            out_v[pl.ds(lax.div(k, 8) * 16, 16)] = zv

        return zv

    lax.fori_loop(0, BPW // 2, body, jnp.zeros((16,), jnp.float32))

    bias_vec = bias_v[...]
    for c in range(BPW // 16):
        z = out_v[pl.ds(c * 16, 16)] + bias_vec
        out_v[pl.ds(c * 16, 16)] = 1.0 / (1.0 + jnp.exp(-z))

    pltpu.sync_copy(out_v, out_hbm.at[pl.ds(base, BPW)])


@jax.jit
def kernel(x, emb_tables, linear_table, bias):
    x_flat = x.reshape(B * F)
    tab = emb_tables.reshape(F * TOTAL, D)
    lin16 = jnp.pad(linear_table.astype(jnp.float32), ((0, 0), (0, 15)))
    bias16 = jnp.broadcast_to(bias.astype(jnp.float32), (16,))

    mesh = plsc.VectorSubcoreMesh(core_axis_name="c", subcore_axis_name="s",
                                  num_cores=NC, num_subcores=NS)
    run = pl.kernel(
        _ffm_body,
        out_type=jax.ShapeDtypeStruct((B,), jnp.float32),
        mesh=mesh,
        compiler_params=pltpu.CompilerParams(use_tc_tiling_on_sc=False),
        scratch_types=[
            pltpu.VMEM((XW + 16,), jnp.int32),
            pltpu.VMEM((16,), jnp.float32),
            pltpu.VMEM((NPAD,), jnp.int32),
            pltpu.VMEM((NPAD,), jnp.int32),
            pltpu.VMEM((NPAD, D), jnp.float32),
            pltpu.VMEM((NPAD, D), jnp.float32),
            pltpu.VMEM((32,), jnp.int32),
            pltpu.VMEM((32,), jnp.int32),
            pltpu.VMEM((32, 16), jnp.float32),
            pltpu.VMEM((32, 16), jnp.float32),
            pltpu.VMEM((BPW,), jnp.float32),
            pltpu.SemaphoreType.DMA,
            pltpu.SemaphoreType.DMA,
        ],
    )
    out = run(x_flat, tab, lin16, bias16)
    return out.reshape(B, 1)
